# baseline (speedup 1.0000x reference)
import functools

import jax
import jax.numpy as jnp
from jax import lax
from jax.experimental import pallas as pl
from jax.experimental.pallas import tpu as pltpu
from jax.experimental.pallas import tpu_sc as plsc

S = 20.0
M = 0.2
LOG2E = 1.4426950408889634
LN2 = 0.6931471805599453


def _make_sc_gather(B, C, NC, NS):
    NW = NC * NS
    per_w = B // NW
    mesh = plsc.VectorSubcoreMesh(core_axis_name="c", subcore_axis_name="s")

    @functools.partial(
        pl.kernel,
        mesh=mesh,
        out_type=jax.ShapeDtypeStruct((B,), jnp.float32),
        scratch_types=[
            pltpu.VMEM((per_w,), jnp.int32),
            pltpu.VMEM((per_w,), jnp.int32),
            pltpu.VMEM((per_w,), jnp.float32),
            pltpu.SemaphoreType.DMA,
        ],
    )
    def sc_gather(flat_hbm, lab_hbm, out_hbm, lab_v, idx_v, t_v, sem):
        wid = lax.axis_index("s") * NC + lax.axis_index("c")
        base = wid * per_w
        pltpu.sync_copy(lab_hbm.at[pl.ds(base, per_w)], lab_v)
        for k in range(per_w // 16):
            sl = pl.ds(k * 16, 16)
            row = base + k * 16 + lax.iota(jnp.int32, 16)
            idx_v[sl] = lab_v[sl] + row * C
        pltpu.async_copy(flat_hbm.at[idx_v], t_v, sem).wait()
        pltpu.sync_copy(t_v, out_hbm.at[pl.ds(base, per_w)])

    return sc_gather


def _stream_body(inp_ref, m_out, s_out, m_s, s_s, *, C, Rb, Cb):
    j = pl.program_id(1)
    nc = pl.num_programs(1)
    K2 = S * LOG2E

    @pl.when(j == 0)
    def _():
        m_s[...] = jnp.full((Rb, 1), -jnp.inf, jnp.float32)
        s_s[...] = jnp.zeros((Rb, 1), jnp.float32)

    def tile(ragged):
        cos = inp_ref[...]
        if ragged:
            lane = lax.broadcasted_iota(jnp.int32, (Rb, Cb), 1)
            rem = C - (C // Cb) * Cb
            cos = jnp.where(lane < rem, cos, -jnp.inf)
        mloc = K2 * jnp.max(cos, axis=1, keepdims=True)
        mold = m_s[...]
        mnew = jnp.maximum(mold, mloc)
        m_s[...] = mnew
        e = jnp.exp2(K2 * cos - mnew)
        s_s[...] = s_s[...] * jnp.exp2(mold - mnew) + jnp.sum(
            e, axis=1, keepdims=True
        )

    @pl.when(j < nc - 1)
    def _():
        tile(False)

    @pl.when(j == nc - 1)
    def _():
        tile(True)

    @pl.when(j == nc - 1)
    def _():
        m_out[...] = m_s[...]
        s_out[...] = s_s[...]


def _combine_body(m_ref, s_ref, t_ref, out_ref, *, B):
    m2 = m_ref[...]
    s = s_ref[...]
    t = t_ref[...]
    mS = m2 * LN2
    a = jnp.exp(S * t - mS)
    b = jnp.exp(S * (t - M) - mS)
    sp = s - a + b
    lse = mS + jnp.log(sp)
    out_ref[0] = jnp.sum(lse - S * (t - M)) / B


@jax.jit
def kernel(input, labels):
    B, C = input.shape
    lab = labels.reshape(B).astype(jnp.int32)

    info = plsc.get_sparse_core_info()
    sc_gather = _make_sc_gather(B, C, info.num_cores, info.num_subcores)
    t = sc_gather(input.reshape(-1), lab)

    Rb = 1024
    Cb = 2048
    nr = B // Rb
    nc = pl.cdiv(C, Cb)
    m2, s = pl.pallas_call(
        functools.partial(_stream_body, C=C, Rb=Rb, Cb=Cb),
        grid=(nr, nc),
        in_specs=[pl.BlockSpec((Rb, Cb), lambda i, j: (i, j))],
        out_specs=[
            pl.BlockSpec((Rb, 1), lambda i, j: (i, 0)),
            pl.BlockSpec((Rb, 1), lambda i, j: (i, 0)),
        ],
        out_shape=[
            jax.ShapeDtypeStruct((B, 1), jnp.float32),
            jax.ShapeDtypeStruct((B, 1), jnp.float32),
        ],
        scratch_shapes=[
            pltpu.VMEM((Rb, 1), jnp.float32),
            pltpu.VMEM((Rb, 1), jnp.float32),
        ],
    )(input)

    out = pl.pallas_call(
        functools.partial(_combine_body, B=B),
        out_specs=pl.BlockSpec(memory_space=pltpu.SMEM),
        out_shape=jax.ShapeDtypeStruct((1,), jnp.float32),
    )(m2, s, t.reshape(B, 1))
    return out[0]


# speedup vs baseline: 2.0986x; 2.0986x over previous
import functools

import jax
import jax.numpy as jnp
from jax import lax
from jax.experimental import pallas as pl
from jax.experimental.pallas import tpu as pltpu

S = 20.0
M = 0.2
LOG2E = 1.4426950408889634
LN2 = 0.6931471805599453


def _body(inp_ref, lab_ref, out_ref, m_s, s_s, t_s, loss_s, *, C, Rb, Cb, B):
    i = pl.program_id(0)
    j = pl.program_id(1)
    nr = pl.num_programs(0)
    nc = pl.num_programs(1)
    K2 = S * LOG2E

    @pl.when(j == 0)
    def _():
        m_s[...] = jnp.full((Rb, 1), -jnp.inf, jnp.float32)
        s_s[...] = jnp.zeros((Rb, 1), jnp.float32)
        t_s[...] = jnp.zeros((Rb, 1), jnp.float32)

    @pl.when((i == 0) & (j == 0))
    def _():
        loss_s[0] = 0.0

    def tile(ragged):
        cos = inp_ref[...]
        lane = lax.broadcasted_iota(jnp.int32, (Rb, Cb), 1)
        islab = lane == (lab_ref[...] - j * Cb)
        t_s[...] += jnp.sum(jnp.where(islab, cos, 0.0), axis=1, keepdims=True)
        if ragged:
            rem = C - (C // Cb) * Cb
            cos = jnp.where(lane < rem, cos, -jnp.inf)
        mloc = K2 * jnp.max(cos, axis=1, keepdims=True)
        mold = m_s[...]
        mnew = jnp.maximum(mold, mloc)
        m_s[...] = mnew
        s_s[...] = s_s[...] * jnp.exp2(mold - mnew) + jnp.sum(
            jnp.exp2(K2 * cos - mnew), axis=1, keepdims=True
        )

    @pl.when(j < nc - 1)
    def _():
        tile(False)

    @pl.when(j == nc - 1)
    def _():
        tile(True)

    @pl.when(j == nc - 1)
    def _():
        m2 = m_s[...]
        t = t_s[...]
        mS = m2 * LN2
        a = jnp.exp(S * t - mS)
        b = jnp.exp(S * (t - M) - mS)
        sp = s_s[...] - a + b
        lse = mS + jnp.log(sp)
        loss_s[0] += jnp.sum(lse - S * (t - M))

    @pl.when((i == nr - 1) & (j == nc - 1))
    def _():
        out_ref[0] = loss_s[0] / B


@jax.jit
def kernel(input, labels):
    B, C = input.shape
    lab = labels.reshape(B, 1).astype(jnp.int32)
    Rb = 1024
    Cb = 2048
    nr = B // Rb
    nc = pl.cdiv(C, Cb)
    out = pl.pallas_call(
        functools.partial(_body, C=C, Rb=Rb, Cb=Cb, B=B),
        grid=(nr, nc),
        in_specs=[
            pl.BlockSpec((Rb, Cb), lambda i, j: (i, j)),
            pl.BlockSpec((Rb, 1), lambda i, j: (i, 0)),
        ],
        out_specs=pl.BlockSpec(memory_space=pltpu.SMEM),
        out_shape=jax.ShapeDtypeStruct((1,), jnp.float32),
        scratch_shapes=[
            pltpu.VMEM((Rb, 1), jnp.float32),
            pltpu.VMEM((Rb, 1), jnp.float32),
            pltpu.VMEM((Rb, 1), jnp.float32),
            pltpu.SMEM((1,), jnp.float32),
        ],
    )(input, lab)
    return out[0]
